# baseline (speedup 1.0000x reference)
import functools

import jax
import jax.numpy as jnp
from jax import lax
from jax.experimental import pallas as pl
from jax.experimental.pallas import tpu as pltpu
from jax.experimental.pallas import tpu_sc as plsc

H = 8
NC = 2
NS = 16
NW = NC * NS
EC = 128
MB = 1000


def _cdiv(a, b):
    return (a + b - 1) // b


def _mesh():
    return plsc.VectorSubcoreMesh(
        core_axis_name="c", subcore_axis_name="s", num_cores=NC, num_subcores=NS
    )


def _zero_shared(acc, buf, sid, rows_per_tile, d):
    z16 = jnp.zeros((16,), jnp.float32)

    def zrow(r, _):
        for j in range(d // 16):
            buf[r, pl.ds(j * 16, 16)] = z16
        return 0

    lax.fori_loop(0, EC, zrow, 0)
    off = 0
    rem = rows_per_tile
    while rem > 0:
        sz = min(EC, rem)
        pltpu.sync_copy(buf.at[pl.ds(0, sz)], acc.at[pl.ds(sid * rows_per_tile + off, sz)])
        off += sz
        rem -= sz


def _dump_shared(acc, buf, out_ref, cid, sid, rows_per_tile):
    off = 0
    rem = rows_per_tile
    while rem > 0:
        sz = min(EC, rem)
        r0 = sid * rows_per_tile + off
        pltpu.sync_copy(acc.at[pl.ds(r0, sz)], buf.at[pl.ds(0, sz)])
        pltpu.sync_copy(buf.at[pl.ds(0, sz)], out_ref.at[cid, pl.ds(r0, sz)])
        off += sz
        rem -= sz


@functools.lru_cache(maxsize=None)
def _sc_pass_a(npad, e_pad):
    nchunks = e_pad // (NW * EC)
    te = nchunks * EC
    rpt = npad // NS

    def body(asrc, adst, esrc2, edst2, ex_out, sp_out, idxs, idxd, bufs, bufd, bufe, acc):
        cid = lax.axis_index("c")
        sid = lax.axis_index("s")
        wid = cid * NS + sid
        _zero_shared(acc, bufe, sid, rpt, 16)
        plsc.subcore_barrier()

        def chunk(k, _):
            base = wid * te + k * EC
            pltpu.sync_copy(esrc2.at[wid * nchunks + k], idxs)
            pltpu.sync_copy(edst2.at[wid * nchunks + k], idxd)
            pltpu.sync_copy(asrc.at[idxs], bufs)
            pltpu.sync_copy(adst.at[idxd], bufd)

            def row(j, _):
                z = bufs[j, :] + bufd[j, :]
                z = jnp.where(z > 0, z, z * jnp.float32(0.2))
                bufe[j, :] = jnp.exp(z)
                return 0

            lax.fori_loop(0, EC, row, 0)
            pltpu.sync_copy(bufe, ex_out.at[pl.ds(base, EC)])
            pltpu.sync_copy(bufe, acc.at[idxd], add=True)
            return 0

        lax.fori_loop(0, nchunks, chunk, 0)
        plsc.subcore_barrier()
        _dump_shared(acc, bufe, sp_out, cid, sid, rpt)

    return pl.kernel(
        body,
        out_type=(
            jax.ShapeDtypeStruct((e_pad, 16), jnp.float32),
            jax.ShapeDtypeStruct((NC, npad, 16), jnp.float32),
        ),
        mesh=_mesh(),
        scratch_types=[
            pltpu.VMEM((EC,), jnp.int32),
            pltpu.VMEM((EC,), jnp.int32),
            pltpu.VMEM((EC, 16), jnp.float32),
            pltpu.VMEM((EC, 16), jnp.float32),
            pltpu.VMEM((EC, 16), jnp.float32),
            pltpu.VMEM_SHARED((npad, 16), jnp.float32),
        ],
        compiler_params=pltpu.CompilerParams(use_tc_tiling_on_sc=False, needs_layout_passes=False),
    )


@functools.lru_cache(maxsize=None)
def _sc_pass_b(npad, e_pad):
    nchunks = e_pad // (NW * EC)
    te = nchunks * EC

    def body(ex, sinv, edst2, al_out, idxd, bufe, bufsv):
        cid = lax.axis_index("c")
        sid = lax.axis_index("s")
        wid = cid * NS + sid

        def chunk(k, _):
            base = wid * te + k * EC
            pltpu.sync_copy(edst2.at[wid * nchunks + k], idxd)
            pltpu.sync_copy(ex.at[pl.ds(base, EC)], bufe)
            pltpu.sync_copy(sinv.at[idxd], bufsv)

            def row(j, _):
                bufe[j, :] = bufe[j, :] * bufsv[j, :]
                return 0

            lax.fori_loop(0, EC, row, 0)
            pltpu.sync_copy(bufe, al_out.at[pl.ds(base, EC)])
            return 0

        lax.fori_loop(0, nchunks, chunk, 0)

    return pl.kernel(
        body,
        out_type=jax.ShapeDtypeStruct((e_pad, 16), jnp.float32),
        mesh=_mesh(),
        scratch_types=[
            pltpu.VMEM((EC,), jnp.int32),
            pltpu.VMEM((EC, 16), jnp.float32),
            pltpu.VMEM((EC, 16), jnp.float32),
        ],
        compiler_params=pltpu.CompilerParams(use_tc_tiling_on_sc=False, needs_layout_passes=False),
    )


@functools.lru_cache(maxsize=None)
def _sc_segsum(nrows, npad, e_pad, d, weighted, npack=1):
    nchunks = e_pad // (NW * EC)
    assert nchunks % 4 == 0 and nchunks >= 8
    ngrp = nchunks // 4
    rpt = npad // NS
    csz = d // npack
    tsz = csz // 16

    def body(*refs):
        if weighted:
            (table, esrc2, edst2, alpha, hb, out,
             ixs0, ixs1, ixs2, ixs3, ixd0, ixd1, ixd2, ixd3,
             buf, ab0, ab1, hb_v, acc,
             si0, si1, si2, si3, sg0, sg1, ss0, ss1) = refs
            ab = [ab0, ab1]
        else:
            (table, esrc2, edst2, out,
             ixs0, ixs1, ixs2, ixs3, ixd0, ixd1, ixd2, ixd3,
             buf, acc,
             si0, si1, si2, si3, sg0, sg1, ss0, ss1) = refs
            alpha = None
            ab = None
        cid = lax.axis_index("c")
        sid = lax.axis_index("s")
        wid = cid * NS + sid
        kbase = wid * nchunks
        _zero_shared(acc, buf.at[0], sid, rpt, d)
        if weighted:
            pltpu.sync_copy(hb, hb_v)
        plsc.subcore_barrier()
        ixs = [ixs0, ixs1, ixs2, ixs3]
        ixd = [ixd0, ixd1, ixd2, ixd3]
        bf = [buf.at[0], buf.at[1]]
        si = [si0, si1, si2, si3]
        sg = [sg0, sg1]
        ss = [ss0, ss1]
        hbv = hb_v[:] if weighted else None

        def idx_start(k, q):
            i1 = pltpu.async_copy(esrc2.at[kbase + k], ixs[q], si[q])
            i2 = pltpu.async_copy(edst2.at[kbase + k], ixd[q], si[q])
            return (i1, i2)

        def g_start(k, p, q):
            g = [pltpu.async_copy(table.at[ixs[q]], bf[p], sg[p])]
            if weighted:
                g.append(pltpu.async_copy(alpha.at[kbase + k], ab[p], sg[p]))
            return g

        def s_start(k, p, q):
            return pltpu.async_copy(bf[p], acc.at[ixd[q]], ss[p], add=True)

        def compute(k, p):
            if not weighted:
                return

            def row(j, _):
                jv = jnp.zeros((16,), jnp.int32) + j
                for tp in range(npack):
                    av = plsc.load_gather(ab[p], [jv, hbv + tp])
                    for t in range(tp * tsz, (tp + 1) * tsz):
                        bf[p][j, pl.ds(t * 16, 16)] = (
                            bf[p][j, pl.ds(t * 16, 16)] * av)
                return 0

            lax.fori_loop(0, EC, row, 0)

        def group(k0):
            ih = [idx_start(k0 + r, r) for r in range(4)]
            for h in ih[0]:
                h.wait()
            gh = [None, None]
            sh = [None, None]
            gh[0] = g_start(k0, 0, 0)
            for r in range(4):
                k = k0 + r
                p = r % 2
                for h in gh[p]:
                    h.wait()
                if sh[(r + 1) % 2] is not None:
                    sh[(r + 1) % 2].wait()
                    sh[(r + 1) % 2] = None
                if r + 1 < 4:
                    for h in ih[r + 1]:
                        h.wait()
                    ih[r + 1] = None
                    gh[(r + 1) % 2] = g_start(k + 1, (r + 1) % 2, r + 1)
                compute(k, p)
                sh[p] = s_start(k, p, r)
            for h in sh:
                if h is not None:
                    h.wait()

        def fbody(kk, _):
            group(kk * 4)
            return 0

        lax.fori_loop(0, ngrp, fbody, 0)
        plsc.subcore_barrier()
        _dump_shared(acc, buf.at[0], out, cid, sid, rpt)

    scratch = [pltpu.VMEM((EC,), jnp.int32)] * 8 + [
        pltpu.VMEM((2, EC, d), jnp.float32),
    ]
    if weighted:
        scratch += [pltpu.VMEM((EC, 16), jnp.float32)] * 2
        scratch += [pltpu.VMEM((16,), jnp.int32)]
    scratch += [
        pltpu.VMEM_SHARED((npad, d), jnp.float32),
    ] + [pltpu.SemaphoreType.DMA] * 8
    return pl.kernel(
        body,
        out_type=jax.ShapeDtypeStruct((NC, npad, d), jnp.float32),
        mesh=_mesh(),
        scratch_types=scratch,
        compiler_params=pltpu.CompilerParams(use_tc_tiling_on_sc=False, needs_layout_passes=False),
    )


def _tc_pre(h, gcn_W, gat_W, wa_s, wa_d, sage_Wr, res_W, res_b):
    n, in_c = h.shape
    out_c = gcn_W.shape[1]
    c8 = gat_W.shape[1]
    grid = n // MB
    has_res = res_W is not None

    def body(*refs):
        if has_res:
            (x, gw, tw, was, wad, swr, rw, rb,
             xw_o, xg_o, as_o, ad_o, sr_o, id_o) = refs
        else:
            (x, gw, tw, was, wad, swr,
             xw_o, xg_o, as_o, ad_o, sr_o) = refs
        xb = x[...]
        f32 = jnp.float32
        xw_o[...] = jnp.dot(xb, gw[...], preferred_element_type=f32)
        xg_o[...] = jnp.dot(xb, tw[...], preferred_element_type=f32)
        z = jnp.zeros((xb.shape[0], 8), f32)
        as_o[...] = jnp.concatenate(
            [jnp.dot(xb, was[...], preferred_element_type=f32), z], axis=1)
        ad_o[...] = jnp.concatenate(
            [jnp.dot(xb, wad[...], preferred_element_type=f32), z], axis=1)
        sr_o[...] = jnp.dot(xb, swr[...], preferred_element_type=f32)
        if has_res:
            id_o[...] = jnp.dot(xb, rw[...], preferred_element_type=f32) + rb[...]

    full = lambda s: pl.BlockSpec(s, lambda m: (0, 0))
    in_specs = [
        pl.BlockSpec((MB, in_c), lambda m: (m, 0)),
        full((in_c, out_c)), full((in_c, c8)),
        full((in_c, 8)), full((in_c, 8)), full((in_c, out_c)),
    ]
    args = [h, gcn_W, gat_W, wa_s, wa_d, sage_Wr]
    outs = [
        jax.ShapeDtypeStruct((n, out_c), jnp.float32),
        jax.ShapeDtypeStruct((n, c8), jnp.float32),
        jax.ShapeDtypeStruct((n, 16), jnp.float32),
        jax.ShapeDtypeStruct((n, 16), jnp.float32),
        jax.ShapeDtypeStruct((n, out_c), jnp.float32),
    ]
    out_specs = [
        pl.BlockSpec((MB, out_c), lambda m: (m, 0)),
        pl.BlockSpec((MB, c8), lambda m: (m, 0)),
        pl.BlockSpec((MB, 16), lambda m: (m, 0)),
        pl.BlockSpec((MB, 16), lambda m: (m, 0)),
        pl.BlockSpec((MB, out_c), lambda m: (m, 0)),
    ]
    if has_res:
        in_specs += [full((in_c, out_c)), full((1, out_c))]
        args += [res_W, res_b.reshape(1, out_c)]
        outs.append(jax.ShapeDtypeStruct((n, out_c), jnp.float32))
        out_specs.append(pl.BlockSpec((MB, out_c), lambda m: (m, 0)))
    return pl.pallas_call(
        body, grid=(grid,), in_specs=in_specs, out_specs=out_specs,
        out_shape=outs,
    )(*args)


def _tc_post(gcn_p, gat_ps, sage_p, h_prev, sage_r, ident, dinv, cntm, scores,
             gcn_b, gat_b, gatT_W, gatT_b, sage_Wl, sage_bl, ln_g, ln_b,
             fus_W, fus_b):
    n, in_c = h_prev.shape
    out_c = gcn_b.shape[0]
    c8 = H * out_c
    ng = len(gat_ps)
    gw = gat_ps[0].shape[2]
    grid = n // MB

    def body(*refs):
        (gp, *rest) = refs
        gs = rest[:ng]
        (sp, hp, sr, idn, dv, cm, sc,
         gb, ab, tw, tb, wl, bl, lg, lb, fw, fb, out) = rest[ng:]
        f32 = jnp.float32
        dot = lambda a, b: jnp.dot(a, b, preferred_element_type=f32)
        gcn = (gp[0] + gp[1]) * dv[...] + gb[...]
        gcn = jnp.maximum(gcn, 0.0)
        gat_cat = jnp.concatenate([g[0] + g[1] for g in gs], axis=1)
        gat = jnp.maximum(gat_cat + ab[...], 0.0)
        gat = dot(gat, tw[...]) + tb[...]
        mean_n = (sp[0] + sp[1] - hp[...]) * cm[...]
        sage = jnp.maximum(dot(mean_n, wl[...]) + bl[...] + sr[...], 0.0)
        s0 = sc[0, 0]
        s1 = sc[0, 1]
        s2 = sc[0, 2]
        merged = s0 * gcn + s1 * gat + s2 * sage
        mu = jnp.mean(merged, axis=1, keepdims=True)
        var = jnp.mean((merged - mu) ** 2, axis=1, keepdims=True)
        merged = (merged - mu) / jnp.sqrt(var + 1e-5) * lg[...] + lb[...]
        idv = idn[...]
        o = dot(merged, fw[0]) + dot(idv, fw[1]) + fb[...]
        out[...] = jnp.maximum(o + idv, 0.0)

    blk = lambda w: pl.BlockSpec((MB, w), lambda m: (m, 0))
    pblk = lambda w: pl.BlockSpec((NC, MB, w), lambda m: (0, m, 0))
    full = lambda s: pl.BlockSpec(s, lambda m: tuple(0 for _ in s))
    in_specs = (
        [pblk(out_c)] + [pblk(gw)] * ng + [pblk(in_c)] +
        [blk(in_c), blk(out_c), blk(out_c), blk(1), blk(1), full((1, 128)),
         full((1, out_c)), full((1, c8)), full((c8, out_c)), full((1, out_c)),
         full((in_c, out_c)), full((1, out_c)), full((1, out_c)),
         full((1, out_c)), full((2, out_c, out_c)), full((1, out_c))]
    )
    args = (
        [gcn_p] + list(gat_ps) + [sage_p, h_prev, sage_r, ident, dinv, cntm,
         scores, gcn_b.reshape(1, -1), gat_b.reshape(1, -1), gatT_W,
         gatT_b.reshape(1, -1), sage_Wl, sage_bl.reshape(1, -1),
         ln_g.reshape(1, -1), ln_b.reshape(1, -1),
         fus_W.reshape(2, out_c, out_c), fus_b.reshape(1, -1)]
    )
    return pl.pallas_call(
        body, grid=(grid,), in_specs=in_specs,
        out_specs=blk(out_c),
        out_shape=jax.ShapeDtypeStruct((n, out_c), jnp.float32),
    )(*args)


def _tc_pool(h, pool_W, pool_b, cls_W1, cls_b1, cls_W2, cls_b2):
    n, d = h.shape
    grid = n // MB
    ncls = cls_W2.shape[1]

    def body(hb, pw, pb, w1, b1, w2, b2, out, s_acc, m_acc):
        i = pl.program_id(0)
        f32 = jnp.float32
        hv = hb[...]
        z = jnp.dot(hv, pw[...], preferred_element_type=f32) + pb[...]
        w = 1.0 / (1.0 + jnp.exp(-z))
        wx = w * hv
        psum = jnp.sum(wx, axis=0, keepdims=True)
        pmax = jnp.max(hv, axis=0, keepdims=True)

        @pl.when(i == 0)
        def _():
            s_acc[...] = psum
            m_acc[...] = pmax

        @pl.when(i > 0)
        def _():
            s_acc[...] = s_acc[...] + psum
            m_acc[...] = jnp.maximum(m_acc[...], pmax)

        @pl.when(i == grid - 1)
        def _():
            pooled = jnp.concatenate(
                [m_acc[...], s_acc[...] * jnp.float32(1.0 / n)], axis=1)
            zc = jnp.dot(pooled, w1[...], preferred_element_type=f32) + b1[...]
            zc = jnp.maximum(zc, 0.0)
            out[...] = jnp.dot(zc, w2[...], preferred_element_type=f32) + b2[...]

    full = lambda s: pl.BlockSpec(s, lambda m: tuple(0 for _ in s))
    return pl.pallas_call(
        body, grid=(grid,),
        in_specs=[
            pl.BlockSpec((MB, d), lambda m: (m, 0)),
            full((d, 1)), full((1, 1)),
            full((2 * d, d)), full((1, d)),
            full((d, ncls)), full((1, ncls)),
        ],
        out_specs=full((1, ncls)),
        out_shape=jax.ShapeDtypeStruct((1, ncls), jnp.float32),
        scratch_shapes=[
            pltpu.VMEM((1, d), jnp.float32),
            pltpu.VMEM((1, d), jnp.float32),
        ],
    )(h, pool_W, pool_b.reshape(1, 1), cls_W1, cls_b1.reshape(1, -1),
      cls_W2, cls_b2.reshape(1, -1))


def _pad_rows(a, npad):
    return jnp.pad(a, ((0, npad - a.shape[0]), (0, 0)))


def _layer(h, h_pad, p, e_src2, e_dst2, npad, e_pad, deg):
    n, in_c = h.shape
    out_c = p['gcn_W'].shape[1]
    c = out_c
    wa_s = jnp.einsum('khc,hc->kh', p['gat_W'].reshape(in_c, H, c),
                      p['gat_att_src'])
    wa_d = jnp.einsum('khc,hc->kh', p['gat_W'].reshape(in_c, H, c),
                      p['gat_att_dst'])
    res_W = p.get('res_W')
    res_b = p.get('res_b')
    pre = _tc_pre(h, p['gcn_W'], p['gat_W'], wa_s, wa_d, p['sage_Wr'],
                  res_W, res_b)
    if res_W is not None:
        xw, xg, a_s, a_d, sage_r, ident = pre
    else:
        xw, xg, a_s, a_d, sage_r = pre
        ident = h

    ex16, s_p = _sc_pass_a(npad, e_pad)(
        _pad_rows(a_s, npad), _pad_rows(a_d, npad), e_src2, e_dst2)
    s_tot = s_p[0] + s_p[1]
    if deg is None:
        deg = s_tot[:n, 8]
    sinv16 = jnp.concatenate(
        [1.0 / (s_tot[:, :8] + 1e-16), jnp.zeros((npad, 8), jnp.float32)],
        axis=1)
    alpha16 = _sc_pass_b(npad, e_pad)(ex16, sinv16, e_dst2)

    dinv = 1.0 / jnp.sqrt(deg)
    xw_pad = _pad_rows(xw * dinv[:, None], npad)
    gcn_p = _sc_segsum(npad, npad, e_pad, out_c, False)(xw_pad, e_src2, e_dst2)

    npack = 128 // c
    ngroups = H // npack
    xg_t = jnp.transpose(xg.reshape(n, ngroups, 128), (1, 0, 2))
    xg_flat = jnp.pad(xg_t, ((0, 0), (0, npad - n), (0, 0))).reshape(
        ngroups * npad, 128)
    alpha3 = alpha16.reshape(e_pad // EC, EC, 16)
    gat_ps = []
    for g in range(ngroups):
        e_src_g = (e_src2 + jnp.int32(g * npad)).reshape(e_pad // EC, EC)
        hb = jnp.full((16,), g * npack, jnp.int32)
        gat_ps.append(
            _sc_segsum(ngroups * npad, npad, e_pad, 128, True, npack)(
                xg_flat, e_src_g, e_dst2, alpha3, hb))

    sage_p = _sc_segsum(npad, npad, e_pad, in_c, False)(h_pad, e_src2, e_dst2)

    cntm = (1.0 / jnp.maximum(deg - 1.0, 1.0))[:, None]
    scores = jnp.zeros((1, 128), jnp.float32).at[0, :3].set(
        jax.nn.softmax(p['attn_w'], axis=0).reshape(3))
    h_next = _tc_post(
        gcn_p, gat_ps, sage_p, h, sage_r, ident, dinv[:, None], cntm, scores,
        p['gcn_b'], p['gat_b'], p['gatT_W'], p['gatT_b'], p['sage_Wl'],
        p['sage_bl'], p['ln_g'], p['ln_b'], p['fus_W'], p['fus_b'])
    return h_next, deg


def kernel(x, edge_index, params):
    n, _ = x.shape
    e = edge_index.shape[1]
    npad = _cdiv(n + 1, NS * 8) * (NS * 8)
    src, dst = edge_index[0], edge_index[1]
    loop = jnp.arange(n, dtype=jnp.int32)
    e_sl = e + n
    e_pad = _cdiv(e_sl, NW * EC * 4) * (NW * EC * 4)
    padv = jnp.full((e_pad - e_sl,), n, jnp.int32)
    e_src = jnp.concatenate([src, loop, padv])
    e_dst = jnp.concatenate([dst, loop, padv])
    e_src2 = e_src.reshape(e_pad // EC, EC)
    e_dst2 = e_dst.reshape(e_pad // EC, EC)

    h = x
    h_pad = _pad_rows(h, npad)
    deg = None
    for name in ('conv1', 'conv2', 'conv3'):
        h, deg = _layer(h, h_pad, params[name], e_src2, e_dst2,
                        npad, e_pad, deg)
        h_pad = _pad_rows(h, npad)

    return _tc_pool(h, params['pool_W'], params['pool_b'], params['cls_W1'],
                    params['cls_b1'], params['cls_W2'], params['cls_b2'])


# speedup vs baseline: 1.0245x; 1.0245x over previous
import functools

import jax
import jax.numpy as jnp
from jax import lax
from jax.experimental import pallas as pl
from jax.experimental.pallas import tpu as pltpu
from jax.experimental.pallas import tpu_sc as plsc

H = 8
NC = 2
NS = 16
NW = NC * NS
EC = 128
MB = 1000


def _cdiv(a, b):
    return (a + b - 1) // b


def _mesh():
    return plsc.VectorSubcoreMesh(
        core_axis_name="c", subcore_axis_name="s", num_cores=NC, num_subcores=NS
    )


def _zero_shared(acc, buf, sid, rows_per_tile, d):
    z16 = jnp.zeros((16,), jnp.float32)

    def zrow(r, _):
        for j in range(d // 16):
            buf[r, pl.ds(j * 16, 16)] = z16
        return 0

    lax.fori_loop(0, EC, zrow, 0)
    off = 0
    rem = rows_per_tile
    while rem > 0:
        sz = min(EC, rem)
        pltpu.sync_copy(buf.at[pl.ds(0, sz)], acc.at[pl.ds(sid * rows_per_tile + off, sz)])
        off += sz
        rem -= sz


def _dump_shared(acc, buf, out_ref, cid, sid, rows_per_tile):
    off = 0
    rem = rows_per_tile
    while rem > 0:
        sz = min(EC, rem)
        r0 = sid * rows_per_tile + off
        pltpu.sync_copy(acc.at[pl.ds(r0, sz)], buf.at[pl.ds(0, sz)])
        pltpu.sync_copy(buf.at[pl.ds(0, sz)], out_ref.at[cid, pl.ds(r0, sz)])
        off += sz
        rem -= sz


@functools.lru_cache(maxsize=None)
def _sc_pass_a(npad, e_pad):
    nchunks = e_pad // (NW * EC)
    te = nchunks * EC
    rpt = npad // NS

    def body(asrc, adst, esrc2, edst2, ex_out, sp_out, idxs, idxd, bufs, bufd, bufe, acc):
        cid = lax.axis_index("c")
        sid = lax.axis_index("s")
        wid = cid * NS + sid
        _zero_shared(acc, bufe, sid, rpt, 16)
        plsc.subcore_barrier()

        def chunk(k, _):
            base = wid * te + k * EC
            pltpu.sync_copy(esrc2.at[wid * nchunks + k], idxs)
            pltpu.sync_copy(edst2.at[wid * nchunks + k], idxd)
            pltpu.sync_copy(asrc.at[idxs], bufs)
            pltpu.sync_copy(adst.at[idxd], bufd)

            def row(j, _):
                z = bufs[j, :] + bufd[j, :]
                z = jnp.where(z > 0, z, z * jnp.float32(0.2))
                bufe[j, :] = jnp.exp(z)
                return 0

            lax.fori_loop(0, EC, row, 0)
            pltpu.sync_copy(bufe, ex_out.at[pl.ds(base, EC)])
            pltpu.sync_copy(bufe, acc.at[idxd], add=True)
            return 0

        lax.fori_loop(0, nchunks, chunk, 0)
        plsc.subcore_barrier()
        _dump_shared(acc, bufe, sp_out, cid, sid, rpt)

    return pl.kernel(
        body,
        out_type=(
            jax.ShapeDtypeStruct((e_pad, 16), jnp.float32),
            jax.ShapeDtypeStruct((NC, npad, 16), jnp.float32),
        ),
        mesh=_mesh(),
        scratch_types=[
            pltpu.VMEM((EC,), jnp.int32),
            pltpu.VMEM((EC,), jnp.int32),
            pltpu.VMEM((EC, 16), jnp.float32),
            pltpu.VMEM((EC, 16), jnp.float32),
            pltpu.VMEM((EC, 16), jnp.float32),
            pltpu.VMEM_SHARED((npad, 16), jnp.float32),
        ],
        compiler_params=pltpu.CompilerParams(use_tc_tiling_on_sc=False, needs_layout_passes=False),
    )


@functools.lru_cache(maxsize=None)
def _sc_pass_b(npad, e_pad):
    nchunks = e_pad // (NW * EC)
    te = nchunks * EC

    def body(ex, sinv, edst2, al_out, idxd, bufe, bufsv):
        cid = lax.axis_index("c")
        sid = lax.axis_index("s")
        wid = cid * NS + sid

        def chunk(k, _):
            base = wid * te + k * EC
            pltpu.sync_copy(edst2.at[wid * nchunks + k], idxd)
            pltpu.sync_copy(ex.at[pl.ds(base, EC)], bufe)
            pltpu.sync_copy(sinv.at[idxd], bufsv)

            def row(j, _):
                bufe[j, :] = bufe[j, :] * bufsv[j, :]
                return 0

            lax.fori_loop(0, EC, row, 0)
            pltpu.sync_copy(bufe, al_out.at[pl.ds(base, EC)])
            return 0

        lax.fori_loop(0, nchunks, chunk, 0)

    return pl.kernel(
        body,
        out_type=jax.ShapeDtypeStruct((e_pad, 16), jnp.float32),
        mesh=_mesh(),
        scratch_types=[
            pltpu.VMEM((EC,), jnp.int32),
            pltpu.VMEM((EC, 16), jnp.float32),
            pltpu.VMEM((EC, 16), jnp.float32),
        ],
        compiler_params=pltpu.CompilerParams(use_tc_tiling_on_sc=False, needs_layout_passes=False),
    )


@functools.lru_cache(maxsize=None)
def _sc_segsum(nrows, npad, e_pad, d, weighted, npack=1):
    nchunks = e_pad // (NW * EC)
    GS = 28
    assert nchunks % GS == 0
    ngrp = nchunks // GS
    rpt = npad // NS
    csz = d // npack
    tsz = csz // 16

    def body(*refs):
        if weighted:
            (table, esrc2, edst2, alpha, hb, out,
             ixs0, ixs1, ixs2, ixs3, ixd0, ixd1, ixd2, ixd3,
             buf, ab0, ab1, hb_v, acc,
             si0, si1, si2, si3, sg0, sg1, ss0, ss1) = refs
            ab = [ab0, ab1]
        else:
            (table, esrc2, edst2, out,
             ixs0, ixs1, ixs2, ixs3, ixd0, ixd1, ixd2, ixd3,
             buf, acc,
             si0, si1, si2, si3, sg0, sg1, ss0, ss1) = refs
            alpha = None
            ab = None
        cid = lax.axis_index("c")
        sid = lax.axis_index("s")
        wid = cid * NS + sid
        kbase = wid * nchunks
        _zero_shared(acc, buf.at[0], sid, rpt, d)
        if weighted:
            pltpu.sync_copy(hb, hb_v)
        plsc.subcore_barrier()
        ixs = [ixs0, ixs1, ixs2, ixs3]
        ixd = [ixd0, ixd1, ixd2, ixd3]
        bf = [buf.at[0], buf.at[1]]
        si = [si0, si1, si2, si3]
        sg = [sg0, sg1]
        ss = [ss0, ss1]
        hbv = hb_v[:] if weighted else None

        def idx_start(k, q):
            i1 = pltpu.async_copy(esrc2.at[kbase + k], ixs[q], si[q])
            i2 = pltpu.async_copy(edst2.at[kbase + k], ixd[q], si[q])
            return (i1, i2)

        def g_start(k, p, q):
            g = [pltpu.async_copy(table.at[ixs[q]], bf[p], sg[p])]
            if weighted:
                g.append(pltpu.async_copy(alpha.at[kbase + k], ab[p], sg[p]))
            return g

        def s_start(k, p, q):
            return pltpu.async_copy(bf[p], acc.at[ixd[q]], ss[p], add=True)

        def compute(k, p):
            if not weighted:
                return

            def row(j, _):
                jv = jnp.zeros((16,), jnp.int32) + j
                for tp in range(npack):
                    av = plsc.load_gather(ab[p], [jv, hbv + tp])
                    for t in range(tp * tsz, (tp + 1) * tsz):
                        bf[p][j, pl.ds(t * 16, 16)] = (
                            bf[p][j, pl.ds(t * 16, 16)] * av)
                return 0

            lax.fori_loop(0, EC, row, 0)

        def group(k0):
            ih = [None, None, None, None]
            for r in range(3):
                ih[r] = idx_start(k0 + r, r)
            for h in ih[0]:
                h.wait()
            ih[0] = None
            gh = [None, None]
            sh = [None, None]
            gh[0] = g_start(k0, 0, 0)
            for r in range(GS):
                k = k0 + r
                p = r % 2
                q = r % 4
                for h in gh[p]:
                    h.wait()
                if sh[(r + 1) % 2] is not None:
                    sh[(r + 1) % 2].wait()
                    sh[(r + 1) % 2] = None
                if r + 3 < GS:
                    ih[(r + 3) % 4] = idx_start(k + 3, (r + 3) % 4)
                if r + 1 < GS:
                    for h in ih[(r + 1) % 4]:
                        h.wait()
                    ih[(r + 1) % 4] = None
                    gh[(r + 1) % 2] = g_start(k + 1, (r + 1) % 2, (r + 1) % 4)
                compute(k, p)
                sh[p] = s_start(k, p, q)
            for h in sh:
                if h is not None:
                    h.wait()

        def fbody(kk, _):
            group(kk * GS)
            return 0

        lax.fori_loop(0, ngrp, fbody, 0)
        plsc.subcore_barrier()
        _dump_shared(acc, buf.at[0], out, cid, sid, rpt)

    scratch = [pltpu.VMEM((EC,), jnp.int32)] * 8 + [
        pltpu.VMEM((2, EC, d), jnp.float32),
    ]
    if weighted:
        scratch += [pltpu.VMEM((EC, 16), jnp.float32)] * 2
        scratch += [pltpu.VMEM((16,), jnp.int32)]
    scratch += [
        pltpu.VMEM_SHARED((npad, d), jnp.float32),
    ] + [pltpu.SemaphoreType.DMA] * 8
    return pl.kernel(
        body,
        out_type=jax.ShapeDtypeStruct((NC, npad, d), jnp.float32),
        mesh=_mesh(),
        scratch_types=scratch,
        compiler_params=pltpu.CompilerParams(use_tc_tiling_on_sc=False, needs_layout_passes=False),
    )


def _tc_pre(h, gcn_W, gat_W, wa_s, wa_d, sage_Wr, res_W, res_b):
    n, in_c = h.shape
    out_c = gcn_W.shape[1]
    c8 = gat_W.shape[1]
    grid = n // MB
    has_res = res_W is not None

    def body(*refs):
        if has_res:
            (x, gw, tw, was, wad, swr, rw, rb,
             xw_o, xg_o, as_o, ad_o, sr_o, id_o) = refs
        else:
            (x, gw, tw, was, wad, swr,
             xw_o, xg_o, as_o, ad_o, sr_o) = refs
        xb = x[...]
        f32 = jnp.float32
        xw_o[...] = jnp.dot(xb, gw[...], preferred_element_type=f32)
        xg_o[...] = jnp.dot(xb, tw[...], preferred_element_type=f32)
        z = jnp.zeros((xb.shape[0], 8), f32)
        as_o[...] = jnp.concatenate(
            [jnp.dot(xb, was[...], preferred_element_type=f32), z], axis=1)
        ad_o[...] = jnp.concatenate(
            [jnp.dot(xb, wad[...], preferred_element_type=f32), z], axis=1)
        sr_o[...] = jnp.dot(xb, swr[...], preferred_element_type=f32)
        if has_res:
            id_o[...] = jnp.dot(xb, rw[...], preferred_element_type=f32) + rb[...]

    full = lambda s: pl.BlockSpec(s, lambda m: (0, 0))
    in_specs = [
        pl.BlockSpec((MB, in_c), lambda m: (m, 0)),
        full((in_c, out_c)), full((in_c, c8)),
        full((in_c, 8)), full((in_c, 8)), full((in_c, out_c)),
    ]
    args = [h, gcn_W, gat_W, wa_s, wa_d, sage_Wr]
    outs = [
        jax.ShapeDtypeStruct((n, out_c), jnp.float32),
        jax.ShapeDtypeStruct((n, c8), jnp.float32),
        jax.ShapeDtypeStruct((n, 16), jnp.float32),
        jax.ShapeDtypeStruct((n, 16), jnp.float32),
        jax.ShapeDtypeStruct((n, out_c), jnp.float32),
    ]
    out_specs = [
        pl.BlockSpec((MB, out_c), lambda m: (m, 0)),
        pl.BlockSpec((MB, c8), lambda m: (m, 0)),
        pl.BlockSpec((MB, 16), lambda m: (m, 0)),
        pl.BlockSpec((MB, 16), lambda m: (m, 0)),
        pl.BlockSpec((MB, out_c), lambda m: (m, 0)),
    ]
    if has_res:
        in_specs += [full((in_c, out_c)), full((1, out_c))]
        args += [res_W, res_b.reshape(1, out_c)]
        outs.append(jax.ShapeDtypeStruct((n, out_c), jnp.float32))
        out_specs.append(pl.BlockSpec((MB, out_c), lambda m: (m, 0)))
    return pl.pallas_call(
        body, grid=(grid,), in_specs=in_specs, out_specs=out_specs,
        out_shape=outs,
    )(*args)


def _tc_post(gcn_p, gat_ps, sage_p, h_prev, sage_r, ident, dinv, cntm, scores,
             gcn_b, gat_b, gatT_W, gatT_b, sage_Wl, sage_bl, ln_g, ln_b,
             fus_W, fus_b):
    n, in_c = h_prev.shape
    out_c = gcn_b.shape[0]
    c8 = H * out_c
    ng = len(gat_ps)
    gw = gat_ps[0].shape[2]
    grid = n // MB

    def body(*refs):
        (gp, *rest) = refs
        gs = rest[:ng]
        (sp, hp, sr, idn, dv, cm, sc,
         gb, ab, tw, tb, wl, bl, lg, lb, fw, fb, out) = rest[ng:]
        f32 = jnp.float32
        dot = lambda a, b: jnp.dot(a, b, preferred_element_type=f32)
        gcn = (gp[0] + gp[1]) * dv[...] + gb[...]
        gcn = jnp.maximum(gcn, 0.0)
        gat_cat = jnp.concatenate([g[0] + g[1] for g in gs], axis=1)
        gat = jnp.maximum(gat_cat + ab[...], 0.0)
        gat = dot(gat, tw[...]) + tb[...]
        mean_n = (sp[0] + sp[1] - hp[...]) * cm[...]
        sage = jnp.maximum(dot(mean_n, wl[...]) + bl[...] + sr[...], 0.0)
        s0 = sc[0, 0]
        s1 = sc[0, 1]
        s2 = sc[0, 2]
        merged = s0 * gcn + s1 * gat + s2 * sage
        mu = jnp.mean(merged, axis=1, keepdims=True)
        var = jnp.mean((merged - mu) ** 2, axis=1, keepdims=True)
        merged = (merged - mu) / jnp.sqrt(var + 1e-5) * lg[...] + lb[...]
        idv = idn[...]
        o = dot(merged, fw[0]) + dot(idv, fw[1]) + fb[...]
        out[...] = jnp.maximum(o + idv, 0.0)

    blk = lambda w: pl.BlockSpec((MB, w), lambda m: (m, 0))
    pblk = lambda w: pl.BlockSpec((NC, MB, w), lambda m: (0, m, 0))
    full = lambda s: pl.BlockSpec(s, lambda m: tuple(0 for _ in s))
    in_specs = (
        [pblk(out_c)] + [pblk(gw)] * ng + [pblk(in_c)] +
        [blk(in_c), blk(out_c), blk(out_c), blk(1), blk(1), full((1, 128)),
         full((1, out_c)), full((1, c8)), full((c8, out_c)), full((1, out_c)),
         full((in_c, out_c)), full((1, out_c)), full((1, out_c)),
         full((1, out_c)), full((2, out_c, out_c)), full((1, out_c))]
    )
    args = (
        [gcn_p] + list(gat_ps) + [sage_p, h_prev, sage_r, ident, dinv, cntm,
         scores, gcn_b.reshape(1, -1), gat_b.reshape(1, -1), gatT_W,
         gatT_b.reshape(1, -1), sage_Wl, sage_bl.reshape(1, -1),
         ln_g.reshape(1, -1), ln_b.reshape(1, -1),
         fus_W.reshape(2, out_c, out_c), fus_b.reshape(1, -1)]
    )
    return pl.pallas_call(
        body, grid=(grid,), in_specs=in_specs,
        out_specs=blk(out_c),
        out_shape=jax.ShapeDtypeStruct((n, out_c), jnp.float32),
    )(*args)


def _tc_pool(h, pool_W, pool_b, cls_W1, cls_b1, cls_W2, cls_b2):
    n, d = h.shape
    grid = n // MB
    ncls = cls_W2.shape[1]

    def body(hb, pw, pb, w1, b1, w2, b2, out, s_acc, m_acc):
        i = pl.program_id(0)
        f32 = jnp.float32
        hv = hb[...]
        z = jnp.dot(hv, pw[...], preferred_element_type=f32) + pb[...]
        w = 1.0 / (1.0 + jnp.exp(-z))
        wx = w * hv
        psum = jnp.sum(wx, axis=0, keepdims=True)
        pmax = jnp.max(hv, axis=0, keepdims=True)

        @pl.when(i == 0)
        def _():
            s_acc[...] = psum
            m_acc[...] = pmax

        @pl.when(i > 0)
        def _():
            s_acc[...] = s_acc[...] + psum
            m_acc[...] = jnp.maximum(m_acc[...], pmax)

        @pl.when(i == grid - 1)
        def _():
            pooled = jnp.concatenate(
                [m_acc[...], s_acc[...] * jnp.float32(1.0 / n)], axis=1)
            zc = jnp.dot(pooled, w1[...], preferred_element_type=f32) + b1[...]
            zc = jnp.maximum(zc, 0.0)
            out[...] = jnp.dot(zc, w2[...], preferred_element_type=f32) + b2[...]

    full = lambda s: pl.BlockSpec(s, lambda m: tuple(0 for _ in s))
    return pl.pallas_call(
        body, grid=(grid,),
        in_specs=[
            pl.BlockSpec((MB, d), lambda m: (m, 0)),
            full((d, 1)), full((1, 1)),
            full((2 * d, d)), full((1, d)),
            full((d, ncls)), full((1, ncls)),
        ],
        out_specs=full((1, ncls)),
        out_shape=jax.ShapeDtypeStruct((1, ncls), jnp.float32),
        scratch_shapes=[
            pltpu.VMEM((1, d), jnp.float32),
            pltpu.VMEM((1, d), jnp.float32),
        ],
    )(h, pool_W, pool_b.reshape(1, 1), cls_W1, cls_b1.reshape(1, -1),
      cls_W2, cls_b2.reshape(1, -1))


def _pad_rows(a, npad):
    return jnp.pad(a, ((0, npad - a.shape[0]), (0, 0)))


def _layer(h, h_pad, p, e_src2, e_dst2, npad, e_pad, deg):
    n, in_c = h.shape
    out_c = p['gcn_W'].shape[1]
    c = out_c
    wa_s = jnp.einsum('khc,hc->kh', p['gat_W'].reshape(in_c, H, c),
                      p['gat_att_src'])
    wa_d = jnp.einsum('khc,hc->kh', p['gat_W'].reshape(in_c, H, c),
                      p['gat_att_dst'])
    res_W = p.get('res_W')
    res_b = p.get('res_b')
    pre = _tc_pre(h, p['gcn_W'], p['gat_W'], wa_s, wa_d, p['sage_Wr'],
                  res_W, res_b)
    if res_W is not None:
        xw, xg, a_s, a_d, sage_r, ident = pre
    else:
        xw, xg, a_s, a_d, sage_r = pre
        ident = h

    ex16, s_p = _sc_pass_a(npad, e_pad)(
        _pad_rows(a_s, npad), _pad_rows(a_d, npad), e_src2, e_dst2)
    s_tot = s_p[0] + s_p[1]
    if deg is None:
        deg = s_tot[:n, 8]
    sinv16 = jnp.concatenate(
        [1.0 / (s_tot[:, :8] + 1e-16), jnp.zeros((npad, 8), jnp.float32)],
        axis=1)
    alpha16 = _sc_pass_b(npad, e_pad)(ex16, sinv16, e_dst2)

    dinv = 1.0 / jnp.sqrt(deg)
    xw_pad = _pad_rows(xw * dinv[:, None], npad)
    gcn_p = _sc_segsum(npad, npad, e_pad, out_c, False)(xw_pad, e_src2, e_dst2)

    npack = 128 // c
    ngroups = H // npack
    xg_t = jnp.transpose(xg.reshape(n, ngroups, 128), (1, 0, 2))
    xg_flat = jnp.pad(xg_t, ((0, 0), (0, npad - n), (0, 0))).reshape(
        ngroups * npad, 128)
    alpha3 = alpha16.reshape(e_pad // EC, EC, 16)
    gat_ps = []
    for g in range(ngroups):
        e_src_g = (e_src2 + jnp.int32(g * npad)).reshape(e_pad // EC, EC)
        hb = jnp.full((16,), g * npack, jnp.int32)
        gat_ps.append(
            _sc_segsum(ngroups * npad, npad, e_pad, 128, True, npack)(
                xg_flat, e_src_g, e_dst2, alpha3, hb))

    sage_p = _sc_segsum(npad, npad, e_pad, in_c, False)(h_pad, e_src2, e_dst2)

    cntm = (1.0 / jnp.maximum(deg - 1.0, 1.0))[:, None]
    scores = jnp.zeros((1, 128), jnp.float32).at[0, :3].set(
        jax.nn.softmax(p['attn_w'], axis=0).reshape(3))
    h_next = _tc_post(
        gcn_p, gat_ps, sage_p, h, sage_r, ident, dinv[:, None], cntm, scores,
        p['gcn_b'], p['gat_b'], p['gatT_W'], p['gatT_b'], p['sage_Wl'],
        p['sage_bl'], p['ln_g'], p['ln_b'], p['fus_W'], p['fus_b'])
    return h_next, deg


def kernel(x, edge_index, params):
    n, _ = x.shape
    e = edge_index.shape[1]
    npad = _cdiv(n + 1, NS * 8) * (NS * 8)
    src, dst = edge_index[0], edge_index[1]
    loop = jnp.arange(n, dtype=jnp.int32)
    e_sl = e + n
    e_pad = _cdiv(e_sl, NW * EC * 28) * (NW * EC * 28)
    padv = jnp.full((e_pad - e_sl,), n, jnp.int32)
    e_src = jnp.concatenate([src, loop, padv])
    e_dst = jnp.concatenate([dst, loop, padv])
    e_src2 = e_src.reshape(e_pad // EC, EC)
    e_dst2 = e_dst.reshape(e_pad // EC, EC)

    h = x
    h_pad = _pad_rows(h, npad)
    deg = None
    for name in ('conv1', 'conv2', 'conv3'):
        h, deg = _layer(h, h_pad, params[name], e_src2, e_dst2,
                        npad, e_pad, deg)
        h_pad = _pad_rows(h, npad)

    return _tc_pool(h, params['pool_W'], params['pool_b'], params['cls_W1'],
                    params['cls_b1'], params['cls_W2'], params['cls_b2'])


# speedup vs baseline: 2.3179x; 2.2624x over previous
import functools

import jax
import jax.numpy as jnp
from jax import lax
from jax.experimental import pallas as pl
from jax.experimental.pallas import tpu as pltpu
from jax.experimental.pallas import tpu_sc as plsc

H = 8
NC = 2
NS = 16
NW = NC * NS
EC = 128
MB = 1000


def _cdiv(a, b):
    return (a + b - 1) // b


def _mesh():
    return plsc.VectorSubcoreMesh(
        core_axis_name="c", subcore_axis_name="s", num_cores=NC, num_subcores=NS
    )


def _zero_shared(acc, buf, sid, rows_per_tile, d):
    z16 = jnp.zeros((16,), jnp.float32)

    def zrow(r, _):
        for j in range(d // 16):
            buf[r, pl.ds(j * 16, 16)] = z16
        return 0

    lax.fori_loop(0, EC, zrow, 0)
    off = 0
    rem = rows_per_tile
    while rem > 0:
        sz = min(EC, rem)
        pltpu.sync_copy(buf.at[pl.ds(0, sz)], acc.at[pl.ds(sid * rows_per_tile + off, sz)])
        off += sz
        rem -= sz


def _dump_shared(acc, buf, out_ref, cid, sid, rows_per_tile):
    off = 0
    rem = rows_per_tile
    while rem > 0:
        sz = min(EC, rem)
        r0 = sid * rows_per_tile + off
        pltpu.sync_copy(acc.at[pl.ds(r0, sz)], buf.at[pl.ds(0, sz)])
        pltpu.sync_copy(buf.at[pl.ds(0, sz)], out_ref.at[cid, pl.ds(r0, sz)])
        off += sz
        rem -= sz


@functools.lru_cache(maxsize=None)
def _sc_pass_a(npad, e_pad):
    nchunks = e_pad // (NW * EC)
    te = nchunks * EC
    rpt = npad // NS

    def body(asrc, adst, esrc2, edst2, ex_out, sp_out, idxs, idxd, bufs, bufd, bufe, acc):
        cid = lax.axis_index("c")
        sid = lax.axis_index("s")
        wid = cid * NS + sid
        _zero_shared(acc, bufe, sid, rpt, 16)
        plsc.subcore_barrier()

        def chunk(k, _):
            base = wid * te + k * EC
            pltpu.sync_copy(esrc2.at[wid * nchunks + k], idxs)
            pltpu.sync_copy(edst2.at[wid * nchunks + k], idxd)
            pltpu.sync_copy(asrc.at[idxs], bufs)
            pltpu.sync_copy(adst.at[idxd], bufd)

            def row(j, _):
                z = bufs[j, :] + bufd[j, :]
                z = jnp.where(z > 0, z, z * jnp.float32(0.2))
                bufe[j, :] = jnp.exp(z)
                return 0

            lax.fori_loop(0, EC, row, 0)
            pltpu.sync_copy(bufe, ex_out.at[pl.ds(base, EC)])
            pltpu.sync_copy(bufe, acc.at[idxd], add=True)
            return 0

        lax.fori_loop(0, nchunks, chunk, 0)
        plsc.subcore_barrier()
        _dump_shared(acc, bufe, sp_out, cid, sid, rpt)

    return pl.kernel(
        body,
        out_type=(
            jax.ShapeDtypeStruct((e_pad, 16), jnp.float32),
            jax.ShapeDtypeStruct((NC, npad, 16), jnp.float32),
        ),
        mesh=_mesh(),
        scratch_types=[
            pltpu.VMEM((EC,), jnp.int32),
            pltpu.VMEM((EC,), jnp.int32),
            pltpu.VMEM((EC, 16), jnp.float32),
            pltpu.VMEM((EC, 16), jnp.float32),
            pltpu.VMEM((EC, 16), jnp.float32),
            pltpu.VMEM_SHARED((npad, 16), jnp.float32),
        ],
        compiler_params=pltpu.CompilerParams(use_tc_tiling_on_sc=False, needs_layout_passes=False),
    )


@functools.lru_cache(maxsize=None)
def _sc_pass_b(npad, e_pad):
    nchunks = e_pad // (NW * EC)
    te = nchunks * EC

    def body(ex, sinv, edst2, al_out, idxd, bufe, bufsv):
        cid = lax.axis_index("c")
        sid = lax.axis_index("s")
        wid = cid * NS + sid

        def chunk(k, _):
            base = wid * te + k * EC
            pltpu.sync_copy(edst2.at[wid * nchunks + k], idxd)
            pltpu.sync_copy(ex.at[pl.ds(base, EC)], bufe)
            pltpu.sync_copy(sinv.at[idxd], bufsv)

            def row(j, _):
                bufe[j, :] = bufe[j, :] * bufsv[j, :]
                return 0

            lax.fori_loop(0, EC, row, 0)
            pltpu.sync_copy(bufe, al_out.at[pl.ds(base, EC)])
            return 0

        lax.fori_loop(0, nchunks, chunk, 0)

    return pl.kernel(
        body,
        out_type=jax.ShapeDtypeStruct((e_pad, 16), jnp.float32),
        mesh=_mesh(),
        scratch_types=[
            pltpu.VMEM((EC,), jnp.int32),
            pltpu.VMEM((EC, 16), jnp.float32),
            pltpu.VMEM((EC, 16), jnp.float32),
        ],
        compiler_params=pltpu.CompilerParams(use_tc_tiling_on_sc=False, needs_layout_passes=False),
    )


@functools.lru_cache(maxsize=None)
def _sc_segsum(nrows, npad, e_pad, d, weighted, npack=1):
    nchunks = e_pad // (NW * EC)
    rpt = npad // NS
    csz = d // npack
    tsz = csz // 16

    def body(*refs):
        if weighted:
            (table, esrc2, edst2, alpha, hb, out,
             ixs0, ixs1, ixs2, ixd0, ixd1, ixd2, buf, ab0, ab1, hb_v, acc,
             si0, si1, si2, sg0, sg1, ss0, ss1) = refs
            ab = [ab0, ab1]
        else:
            (table, esrc2, edst2, out,
             ixs0, ixs1, ixs2, ixd0, ixd1, ixd2, buf, acc,
             si0, si1, si2, sg0, sg1, ss0, ss1) = refs
            alpha = None
            ab = None
        cid = lax.axis_index("c")
        sid = lax.axis_index("s")
        wid = cid * NS + sid
        kbase = wid * nchunks
        _zero_shared(acc, buf.at[0], sid, rpt, d)
        if weighted:
            pltpu.sync_copy(hb, hb_v)
        plsc.subcore_barrier()
        ixs = [ixs0, ixs1, ixs2]
        ixd = [ixd0, ixd1, ixd2]
        bf = [buf.at[0], buf.at[1]]
        si = [si0, si1, si2]
        sg = [sg0, sg1]
        ss = [ss0, ss1]
        ih = [None, None, None]
        gh = [None, None]
        sh = [None, None]
        hbv = hb_v[:] if weighted else None

        def start_idx(k):
            q = k % 3
            i1 = pltpu.async_copy(esrc2.at[kbase + k], ixs[q], si[q])
            i2 = pltpu.async_copy(edst2.at[kbase + k], ixd[q], si[q])
            ih[q] = (i1, i2)

        def start_gather(k):
            p = k % 2
            q = k % 3
            g = [pltpu.async_copy(table.at[ixs[q]], bf[p], sg[p])]
            if weighted:
                g.append(pltpu.async_copy(alpha.at[kbase + k], ab[p], sg[p]))
            gh[p] = g

        start_idx(0)
        if nchunks > 1:
            start_idx(1)
        for h in ih[0]:
            h.wait()
        ih[0] = None
        start_gather(0)
        for k in range(nchunks):
            p = k % 2
            for h in gh[p]:
                h.wait()
            if sh[(k + 1) % 2] is not None:
                sh[(k + 1) % 2].wait()
                sh[(k + 1) % 2] = None
            if k + 2 < nchunks:
                start_idx(k + 2)
            if k + 1 < nchunks:
                q1 = (k + 1) % 3
                if ih[q1] is not None:
                    for h in ih[q1]:
                        h.wait()
                    ih[q1] = None
                start_gather(k + 1)
            if weighted:

                def row(j, _, p=p):
                    jv = jnp.zeros((16,), jnp.int32) + j
                    for tp in range(npack):
                        av = plsc.load_gather(ab[p], [jv, hbv + tp])
                        for t in range(tp * tsz, (tp + 1) * tsz):
                            bf[p][j, pl.ds(t * 16, 16)] = (
                                bf[p][j, pl.ds(t * 16, 16)] * av)
                    return 0

                lax.fori_loop(0, EC, row, 0)
            sh[p] = pltpu.async_copy(bf[p], acc.at[ixd[k % 3]], ss[p], add=True)
        for h in sh:
            if h is not None:
                h.wait()
        plsc.subcore_barrier()
        _dump_shared(acc, buf.at[0], out, cid, sid, rpt)

    scratch = [pltpu.VMEM((EC,), jnp.int32)] * 6 + [
        pltpu.VMEM((2, EC, d), jnp.float32),
    ]
    if weighted:
        scratch += [pltpu.VMEM((EC, 16), jnp.float32)] * 2
        scratch += [pltpu.VMEM((16,), jnp.int32)]
    scratch += [
        pltpu.VMEM_SHARED((npad, d), jnp.float32),
    ] + [pltpu.SemaphoreType.DMA] * 7
    return pl.kernel(
        body,
        out_type=jax.ShapeDtypeStruct((NC, npad, d), jnp.float32),
        mesh=_mesh(),
        scratch_types=scratch,
        compiler_params=pltpu.CompilerParams(use_tc_tiling_on_sc=False, needs_layout_passes=False),
    )


def _tc_pre(h, gcn_W, gat_W, wa_s, wa_d, sage_Wr, res_W, res_b):
    n, in_c = h.shape
    out_c = gcn_W.shape[1]
    c8 = gat_W.shape[1]
    grid = n // MB
    has_res = res_W is not None

    def body(*refs):
        if has_res:
            (x, gw, tw, was, wad, swr, rw, rb,
             xw_o, xg_o, as_o, ad_o, sr_o, id_o) = refs
        else:
            (x, gw, tw, was, wad, swr,
             xw_o, xg_o, as_o, ad_o, sr_o) = refs
        xb = x[...]
        f32 = jnp.float32
        xw_o[...] = jnp.dot(xb, gw[...], preferred_element_type=f32)
        xg_o[...] = jnp.dot(xb, tw[...], preferred_element_type=f32)
        z = jnp.zeros((xb.shape[0], 8), f32)
        as_o[...] = jnp.concatenate(
            [jnp.dot(xb, was[...], preferred_element_type=f32), z], axis=1)
        ad_o[...] = jnp.concatenate(
            [jnp.dot(xb, wad[...], preferred_element_type=f32), z], axis=1)
        sr_o[...] = jnp.dot(xb, swr[...], preferred_element_type=f32)
        if has_res:
            id_o[...] = jnp.dot(xb, rw[...], preferred_element_type=f32) + rb[...]

    full = lambda s: pl.BlockSpec(s, lambda m: (0, 0))
    in_specs = [
        pl.BlockSpec((MB, in_c), lambda m: (m, 0)),
        full((in_c, out_c)), full((in_c, c8)),
        full((in_c, 8)), full((in_c, 8)), full((in_c, out_c)),
    ]
    args = [h, gcn_W, gat_W, wa_s, wa_d, sage_Wr]
    outs = [
        jax.ShapeDtypeStruct((n, out_c), jnp.float32),
        jax.ShapeDtypeStruct((n, c8), jnp.float32),
        jax.ShapeDtypeStruct((n, 16), jnp.float32),
        jax.ShapeDtypeStruct((n, 16), jnp.float32),
        jax.ShapeDtypeStruct((n, out_c), jnp.float32),
    ]
    out_specs = [
        pl.BlockSpec((MB, out_c), lambda m: (m, 0)),
        pl.BlockSpec((MB, c8), lambda m: (m, 0)),
        pl.BlockSpec((MB, 16), lambda m: (m, 0)),
        pl.BlockSpec((MB, 16), lambda m: (m, 0)),
        pl.BlockSpec((MB, out_c), lambda m: (m, 0)),
    ]
    if has_res:
        in_specs += [full((in_c, out_c)), full((1, out_c))]
        args += [res_W, res_b.reshape(1, out_c)]
        outs.append(jax.ShapeDtypeStruct((n, out_c), jnp.float32))
        out_specs.append(pl.BlockSpec((MB, out_c), lambda m: (m, 0)))
    return pl.pallas_call(
        body, grid=(grid,), in_specs=in_specs, out_specs=out_specs,
        out_shape=outs,
    )(*args)


def _tc_post(gcn_p, gat_ps, sage_p, h_prev, sage_r, ident, dinv, cntm, scores,
             gcn_b, gat_b, gatT_W, gatT_b, sage_Wl, sage_bl, ln_g, ln_b,
             fus_W, fus_b):
    n, in_c = h_prev.shape
    out_c = gcn_b.shape[0]
    c8 = H * out_c
    ng = len(gat_ps)
    gw = gat_ps[0].shape[2]
    grid = n // MB

    def body(*refs):
        (gp, *rest) = refs
        gs = rest[:ng]
        (sp, hp, sr, idn, dv, cm, sc,
         gb, ab, tw, tb, wl, bl, lg, lb, fw, fb, out) = rest[ng:]
        f32 = jnp.float32
        dot = lambda a, b: jnp.dot(a, b, preferred_element_type=f32)
        gcn = (gp[0] + gp[1]) * dv[...] + gb[...]
        gcn = jnp.maximum(gcn, 0.0)
        gat_cat = jnp.concatenate([g[0] + g[1] for g in gs], axis=1)
        gat = jnp.maximum(gat_cat + ab[...], 0.0)
        gat = dot(gat, tw[...]) + tb[...]
        mean_n = (sp[0] + sp[1] - hp[...]) * cm[...]
        sage = jnp.maximum(dot(mean_n, wl[...]) + bl[...] + sr[...], 0.0)
        s0 = sc[0, 0]
        s1 = sc[0, 1]
        s2 = sc[0, 2]
        merged = s0 * gcn + s1 * gat + s2 * sage
        mu = jnp.mean(merged, axis=1, keepdims=True)
        var = jnp.mean((merged - mu) ** 2, axis=1, keepdims=True)
        merged = (merged - mu) / jnp.sqrt(var + 1e-5) * lg[...] + lb[...]
        idv = idn[...]
        o = dot(merged, fw[0]) + dot(idv, fw[1]) + fb[...]
        out[...] = jnp.maximum(o + idv, 0.0)

    blk = lambda w: pl.BlockSpec((MB, w), lambda m: (m, 0))
    pblk = lambda w: pl.BlockSpec((NC, MB, w), lambda m: (0, m, 0))
    full = lambda s: pl.BlockSpec(s, lambda m: tuple(0 for _ in s))
    in_specs = (
        [pblk(out_c)] + [pblk(gw)] * ng + [pblk(in_c)] +
        [blk(in_c), blk(out_c), blk(out_c), blk(1), blk(1), full((1, 128)),
         full((1, out_c)), full((1, c8)), full((c8, out_c)), full((1, out_c)),
         full((in_c, out_c)), full((1, out_c)), full((1, out_c)),
         full((1, out_c)), full((2, out_c, out_c)), full((1, out_c))]
    )
    args = (
        [gcn_p] + list(gat_ps) + [sage_p, h_prev, sage_r, ident, dinv, cntm,
         scores, gcn_b.reshape(1, -1), gat_b.reshape(1, -1), gatT_W,
         gatT_b.reshape(1, -1), sage_Wl, sage_bl.reshape(1, -1),
         ln_g.reshape(1, -1), ln_b.reshape(1, -1),
         fus_W.reshape(2, out_c, out_c), fus_b.reshape(1, -1)]
    )
    return pl.pallas_call(
        body, grid=(grid,), in_specs=in_specs,
        out_specs=blk(out_c),
        out_shape=jax.ShapeDtypeStruct((n, out_c), jnp.float32),
    )(*args)


def _tc_pool(h, pool_W, pool_b, cls_W1, cls_b1, cls_W2, cls_b2):
    n, d = h.shape
    grid = n // MB
    ncls = cls_W2.shape[1]

    def body(hb, pw, pb, w1, b1, w2, b2, out, s_acc, m_acc):
        i = pl.program_id(0)
        f32 = jnp.float32
        hv = hb[...]
        z = jnp.dot(hv, pw[...], preferred_element_type=f32) + pb[...]
        w = 1.0 / (1.0 + jnp.exp(-z))
        wx = w * hv
        psum = jnp.sum(wx, axis=0, keepdims=True)
        pmax = jnp.max(hv, axis=0, keepdims=True)

        @pl.when(i == 0)
        def _():
            s_acc[...] = psum
            m_acc[...] = pmax

        @pl.when(i > 0)
        def _():
            s_acc[...] = s_acc[...] + psum
            m_acc[...] = jnp.maximum(m_acc[...], pmax)

        @pl.when(i == grid - 1)
        def _():
            pooled = jnp.concatenate(
                [m_acc[...], s_acc[...] * jnp.float32(1.0 / n)], axis=1)
            zc = jnp.dot(pooled, w1[...], preferred_element_type=f32) + b1[...]
            zc = jnp.maximum(zc, 0.0)
            out[...] = jnp.dot(zc, w2[...], preferred_element_type=f32) + b2[...]

    full = lambda s: pl.BlockSpec(s, lambda m: tuple(0 for _ in s))
    return pl.pallas_call(
        body, grid=(grid,),
        in_specs=[
            pl.BlockSpec((MB, d), lambda m: (m, 0)),
            full((d, 1)), full((1, 1)),
            full((2 * d, d)), full((1, d)),
            full((d, ncls)), full((1, ncls)),
        ],
        out_specs=full((1, ncls)),
        out_shape=jax.ShapeDtypeStruct((1, ncls), jnp.float32),
        scratch_shapes=[
            pltpu.VMEM((1, d), jnp.float32),
            pltpu.VMEM((1, d), jnp.float32),
        ],
    )(h, pool_W, pool_b.reshape(1, 1), cls_W1, cls_b1.reshape(1, -1),
      cls_W2, cls_b2.reshape(1, -1))


def _pad_rows(a, npad):
    return jnp.pad(a, ((0, npad - a.shape[0]), (0, 0)))


def _layer(h, h_pad, p, e_src2, e_dst2, npad, e_pad, deg):
    n, in_c = h.shape
    out_c = p['gcn_W'].shape[1]
    c = out_c
    wa_s = jnp.einsum('khc,hc->kh', p['gat_W'].reshape(in_c, H, c),
                      p['gat_att_src'])
    wa_d = jnp.einsum('khc,hc->kh', p['gat_W'].reshape(in_c, H, c),
                      p['gat_att_dst'])
    res_W = p.get('res_W')
    res_b = p.get('res_b')
    pre = _tc_pre(h, p['gcn_W'], p['gat_W'], wa_s, wa_d, p['sage_Wr'],
                  res_W, res_b)
    if res_W is not None:
        xw, xg, a_s, a_d, sage_r, ident = pre
    else:
        xw, xg, a_s, a_d, sage_r = pre
        ident = h

    ex16, s_p = _sc_pass_a(npad, e_pad)(
        _pad_rows(a_s, npad), _pad_rows(a_d, npad), e_src2, e_dst2)
    s_tot = s_p[0] + s_p[1]
    if deg is None:
        deg = s_tot[:n, 8]
    sinv16 = jnp.concatenate(
        [1.0 / (s_tot[:, :8] + 1e-16), jnp.zeros((npad, 8), jnp.float32)],
        axis=1)
    alpha16 = _sc_pass_b(npad, e_pad)(ex16, sinv16, e_dst2)

    dinv = 1.0 / jnp.sqrt(deg)
    xw_pad = _pad_rows(xw * dinv[:, None], npad)
    gcn_p = _sc_segsum(npad, npad, e_pad, out_c, False)(xw_pad, e_src2, e_dst2)

    npack = min(2, 128 // c)
    ngroups = H // npack
    dpack = npack * c
    xg_t = jnp.transpose(xg.reshape(n, ngroups, dpack), (1, 0, 2))
    xg_flat = jnp.pad(xg_t, ((0, 0), (0, npad - n), (0, 0))).reshape(
        ngroups * npad, dpack)
    alpha3 = alpha16.reshape(e_pad // EC, EC, 16)
    gat_ps = []
    for g in range(ngroups):
        e_src_g = (e_src2 + jnp.int32(g * npad)).reshape(e_pad // EC, EC)
        hb = jnp.full((16,), g * npack, jnp.int32)
        gat_ps.append(
            _sc_segsum(ngroups * npad, npad, e_pad, dpack, True, npack)(
                xg_flat, e_src_g, e_dst2, alpha3, hb))

    sage_p = _sc_segsum(npad, npad, e_pad, in_c, False)(h_pad, e_src2, e_dst2)

    cntm = (1.0 / jnp.maximum(deg - 1.0, 1.0))[:, None]
    scores = jnp.zeros((1, 128), jnp.float32).at[0, :3].set(
        jax.nn.softmax(p['attn_w'], axis=0).reshape(3))
    h_next = _tc_post(
        gcn_p, gat_ps, sage_p, h, sage_r, ident, dinv[:, None], cntm, scores,
        p['gcn_b'], p['gat_b'], p['gatT_W'], p['gatT_b'], p['sage_Wl'],
        p['sage_bl'], p['ln_g'], p['ln_b'], p['fus_W'], p['fus_b'])
    return h_next, deg


def kernel(x, edge_index, params):
    n, _ = x.shape
    e = edge_index.shape[1]
    npad = _cdiv(n + 1, NS * 8) * (NS * 8)
    src, dst = edge_index[0], edge_index[1]
    loop = jnp.arange(n, dtype=jnp.int32)
    e_sl = e + n
    e_pad = _cdiv(e_sl, NW * EC) * (NW * EC)
    padv = jnp.full((e_pad - e_sl,), n, jnp.int32)
    e_src = jnp.concatenate([src, loop, padv])
    e_dst = jnp.concatenate([dst, loop, padv])
    e_src2 = e_src.reshape(e_pad // EC, EC)
    e_dst2 = e_dst.reshape(e_pad // EC, EC)

    h = x
    h_pad = _pad_rows(h, npad)
    deg = None
    for name in ('conv1', 'conv2', 'conv3'):
        h, deg = _layer(h, h_pad, params[name], e_src2, e_dst2,
                        npad, e_pad, deg)
        h_pad = _pad_rows(h, npad)

    return _tc_pool(h, params['pool_W'], params['pool_b'], params['cls_W1'],
                    params['cls_b1'], params['cls_W2'], params['cls_b2'])
